# Initial kernel scaffold; baseline (speedup 1.0000x reference)
import jax
import jax.numpy as jnp
from jax.experimental import pallas as pl


def kernel(u_features, v_features, support_rows, support_cols, support_vals, u_indices, v_indices, W_gcn, W_dense, P, coeffs):
    raise NotImplementedError("write your pallas kernel here")


# speedup vs baseline: 5.1132x; 5.1132x over previous
import functools

import jax
import jax.numpy as jnp
from jax import lax
from jax.experimental import pallas as pl
from jax.experimental.pallas import tpu as pltpu
from jax.experimental.pallas import tpu_sc as plsc

_NU = 10000
_NUP = 10240
_PF = 128
_NP = 10
_EP = 131072
_NCH = 32
_CH = 128
_LP = 163840
_LPW = 5120
_DNCH = 40

_GD = lax.GatherDimensionNumbers(
    offset_dims=(), collapsed_slice_dims=(0,), start_index_map=(0,))


def _bcast_lane(vec, lane):
    idx = jnp.full((16, 1), lane, jnp.int32)
    return lax.gather(vec, idx, _GD, slice_sizes=(1,),
                      mode=lax.GatherScatterMode.PROMISE_IN_BOUNDS)


def _agg_body(tables, src_hbm, dst_hbm, vals_hbm, out_hbm,
              src_v, dst_v, vals_v, rows_v, zeros_v, accum, sem):
    c = lax.axis_index("c")
    s = lax.axis_index("s")
    wid = c * 16 + s
    row0 = s * 640

    z16 = jnp.zeros((16,), jnp.float32)

    def zfill(i, carry):
        zeros_v[i // 8, pl.ds((i % 8) * 16, 16)] = z16
        return carry

    lax.fori_loop(0, _CH * 8, zfill, 0)

    def one_pass(p, carry):
        for b in range(5):
            pltpu.sync_copy(zeros_v, accum.at[pl.ds(row0 + b * _CH, _CH)])
        plsc.subcore_barrier()
        pltpu.sync_copy(src_hbm.at[p, wid], src_v)
        pltpu.sync_copy(dst_hbm.at[p, wid], dst_v)
        pltpu.sync_copy(vals_hbm.at[p, wid], vals_v)

        def one_chunk(ch, carry2):
            pltpu.async_copy(tables.at[src_v.at[ch]], rows_v, sem).wait()

            def scale(g, carry3):
                valv = vals_v[ch, pl.ds(g * 16, 16)]
                for e16 in range(16):
                    vv = _bcast_lane(valv, e16)
                    e = g * 16 + e16
                    for j in range(8):
                        rows_v[e, pl.ds(j * 16, 16)] = (
                            rows_v[e, pl.ds(j * 16, 16)] * vv)
                return carry3

            lax.fori_loop(0, _CH // 16, scale, 0)
            pltpu.sync_copy(rows_v, accum.at[dst_v.at[ch]], add=True)
            return carry2

        lax.fori_loop(0, _NCH, one_chunk, 0)
        plsc.subcore_barrier()
        for b in range(5):
            pltpu.sync_copy(
                accum.at[pl.ds(row0 + b * _CH, _CH)],
                out_hbm.at[c, p, pl.ds(row0 + b * _CH, _CH)])
        return carry

    lax.fori_loop(0, _NP, one_pass, 0)


def _dec_body(qcat, vtab, uidx, vidx, cvec, out_hbm,
              uix_v, vix_v, q_v, v_v, o_v, c_v, sem, sem2):
    c = lax.axis_index("c")
    s = lax.axis_index("s")
    wid = c * 16 + s
    pltpu.sync_copy(uidx.at[wid], uix_v)
    pltpu.sync_copy(vidx.at[wid], vix_v)
    pltpu.sync_copy(cvec, c_v)
    c0 = c_v[pl.ds(0, 16)]
    c1 = c_v[pl.ds(16, 16)]

    def one_chunk(ch, carry):
        cp1 = pltpu.async_copy(qcat.at[uix_v.at[ch]], q_v, sem)
        cp2 = pltpu.async_copy(vtab.at[vix_v.at[ch]], v_v, sem2)
        cp1.wait()
        cp2.wait()

        def link(e, carry2):
            vs0 = v_v[e, pl.ds(0, 16)]
            acc0 = q_v[e, pl.ds(0, 16)] * vs0
            acc1 = q_v[e, pl.ds(_PF, 16)] * vs0
            for j in range(1, 5):
                vsj = v_v[e, pl.ds(16 * j, 16)]
                acc0 = acc0 + q_v[e, pl.ds(16 * j, 16)] * vsj
                acc1 = acc1 + q_v[e, pl.ds(_PF + 16 * j, 16)] * vsj
            b0 = jnp.sum(acc0)
            b1 = jnp.sum(acc1)
            o_v[e // 8, pl.ds((e % 8) * 16, 16)] = b0 * c0 + b1 * c1
            return carry2

        lax.fori_loop(0, _CH, link, 0)
        pltpu.sync_copy(o_v, out_hbm.at[pl.ds(wid * 640 + ch * 16, 16)])
        return carry

    lax.fori_loop(0, _DNCH, one_chunk, 0)


@functools.cache
def _build_sc_kernels():
    mesh = plsc.VectorSubcoreMesh(core_axis_name="c", subcore_axis_name="s",
                                  num_cores=2, num_subcores=16)
    params = pltpu.CompilerParams(needs_layout_passes=False)
    agg = pl.kernel(
        _agg_body,
        out_type=jax.ShapeDtypeStruct((2, _NP, _NUP, _PF), jnp.float32),
        mesh=mesh,
        compiler_params=params,
        scratch_types=[
            pltpu.VMEM((_NCH, _CH), jnp.int32),
            pltpu.VMEM((_NCH, _CH), jnp.int32),
            pltpu.VMEM((_NCH, _CH), jnp.float32),
            pltpu.VMEM((_CH, _PF), jnp.float32),
            pltpu.VMEM((_CH, _PF), jnp.float32),
            pltpu.VMEM_SHARED((_NUP, _PF), jnp.float32),
            pltpu.SemaphoreType.DMA,
        ],
    )
    dec = pl.kernel(
        _dec_body,
        out_type=jax.ShapeDtypeStruct((_LP // 8, _PF), jnp.float32),
        mesh=mesh,
        compiler_params=params,
        scratch_types=[
            pltpu.VMEM((_DNCH, _CH), jnp.int32),
            pltpu.VMEM((_DNCH, _CH), jnp.int32),
            pltpu.VMEM((_CH, 2 * _PF), jnp.float32),
            pltpu.VMEM((_CH, _PF), jnp.float32),
            pltpu.VMEM((16, _PF), jnp.float32),
            pltpu.VMEM((32,), jnp.float32),
            pltpu.SemaphoreType.DMA,
            pltpu.SemaphoreType.DMA,
        ],
    )
    return agg, dec


def _mm1_body(uv_ref, w_ref, o_ref):
    o_ref[0] = jnp.dot(uv_ref[0], w_ref[0],
                       preferred_element_type=jnp.float32)


_tc_tables = pl.pallas_call(
    _mm1_body,
    grid=(10, 10),
    in_specs=[
        pl.BlockSpec((1, 1000, 128), lambda p, m: (p % 2, m, 0)),
        pl.BlockSpec((1, 128, _PF), lambda p, m: (p, 0, 0)),
    ],
    out_specs=pl.BlockSpec((1, 1000, _PF), lambda p, m: (p, m, 0)),
    out_shape=jax.ShapeDtypeStruct((_NP, _NU, _PF), jnp.float32),
)


def _dense_body(a0_ref, a1_ref, wd_ref, o_ref):
    r = pl.program_id(2)
    z = jnp.maximum(a0_ref[0] + a1_ref[0], 0.0)
    part = jnp.dot(z, wd_ref[0], preferred_element_type=jnp.float32)

    @pl.when(r == 0)
    def _():
        o_ref[0] = part

    @pl.when(r != 0)
    def _():
        o_ref[0] = o_ref[0] + part


_tc_dense = pl.pallas_call(
    _dense_body,
    grid=(2, 10, 5),
    in_specs=[
        pl.BlockSpec((1, 1024, _PF), lambda d, m, r: (2 * r + d, m, 0)),
        pl.BlockSpec((1, 1024, _PF), lambda d, m, r: (2 * r + d, m, 0)),
        pl.BlockSpec((1, _PF, _PF), lambda d, m, r: (r, 0, 0)),
    ],
    out_specs=pl.BlockSpec((1, 1024, _PF), lambda d, m, r: (d, m, 0)),
    out_shape=jax.ShapeDtypeStruct((2, _NUP, _PF), jnp.float32),
)


def _q_body(h_ref, p_ref, o_ref):
    o_ref[...] = jnp.dot(h_ref[...], p_ref[...],
                         preferred_element_type=jnp.float32)


_tc_q = pl.pallas_call(
    _q_body,
    grid=(10,),
    in_specs=[
        pl.BlockSpec((1024, _PF), lambda m: (m, 0)),
        pl.BlockSpec((_PF, 2 * _PF), lambda m: (0, 0)),
    ],
    out_specs=pl.BlockSpec((1024, 2 * _PF), lambda m: (m, 0)),
    out_shape=jax.ShapeDtypeStruct((_NUP, 2 * _PF), jnp.float32),
)


def kernel(u_features, v_features, support_rows, support_cols, support_vals,
           u_indices, v_indices, W_gcn, W_dense, P, coeffs):
    f32 = jnp.float32
    i32 = jnp.int32

    w5 = jnp.pad(W_gcn.reshape(128, 5, 100).transpose(1, 0, 2),
                 ((0, 0), (0, 0), (0, _PF - 100)))
    w_stack = jnp.repeat(w5, 2, axis=0)
    uv = jnp.stack([v_features, u_features])
    wd_stack = jnp.pad(W_dense.reshape(5, 100, 75),
                       ((0, 0), (0, _PF - 100), (0, _PF - 75)))
    p0 = jnp.pad(P[0], ((0, _PF - 75), (0, _PF - 75)))
    p1 = jnp.pad(P[1], ((0, _PF - 75), (0, _PF - 75)))
    pcat = jnp.concatenate([p0, p1], axis=1)
    cvec = jnp.concatenate([
        coeffs[0], jnp.zeros((11,), f32),
        coeffs[1], jnp.zeros((11,), f32)])

    pad_e = (jnp.arange(_EP - 128000, dtype=i32) * 97) % _NU
    zval = jnp.zeros((_EP - 128000,), f32)
    srcs, dsts, valsl = [], [], []
    for r in range(5):
        for d in range(2):
            p = 2 * r + d
            src = support_cols[r] if d == 0 else support_rows[r]
            dst = support_rows[r] if d == 0 else support_cols[r]
            srcs.append(jnp.concatenate([src, pad_e]) + p * _NU)
            dsts.append(jnp.concatenate([dst, pad_e]))
            valsl.append(jnp.concatenate([support_vals[r], zval]))
    src_hbm = jnp.stack(srcs).reshape(_NP, 32, _NCH, _CH)
    dst_hbm = jnp.stack(dsts).reshape(_NP, 32, _NCH, _CH)
    vals_hbm = jnp.stack(valsl).reshape(_NP, 32, _NCH, _CH)

    pad_l = (jnp.arange(_LP - 160000, dtype=i32) * 131) % _NU
    uix = jnp.concatenate([u_indices, pad_l]).reshape(32, _DNCH, _CH)
    vix = jnp.concatenate([v_indices, pad_l]).reshape(32, _DNCH, _CH)

    sc_aggregate, sc_decoder = _build_sc_kernels()
    tables = _tc_tables(uv, w_stack).reshape(_NP * _NU, _PF)
    aggs = sc_aggregate(tables, src_hbm, dst_hbm, vals_hbm)
    h = _tc_dense(aggs[0], aggs[1], wd_stack)
    qcat = _tc_q(h[0], pcat)
    out = sc_decoder(qcat, h[1], uix, vix, cvec)
    return out.reshape(_LP, 16)[:160000, :5]


# speedup vs baseline: 7.1991x; 1.4080x over previous
import functools

import jax
import jax.numpy as jnp
from jax import lax
from jax.experimental import pallas as pl
from jax.experimental.pallas import tpu as pltpu
from jax.experimental.pallas import tpu_sc as plsc

_NU = 10000
_NUP = 10240
_PF = 128
_NP = 10
_EP = 131072
_NCH = 32
_CH = 128
_LP = 163840
_LPW = 5120
_DNCH = 40

_GD = lax.GatherDimensionNumbers(
    offset_dims=(), collapsed_slice_dims=(0,), start_index_map=(0,))


def _bcast_lane(vec, lane):
    idx = jnp.full((16, 1), lane, jnp.int32)
    return lax.gather(vec, idx, _GD, slice_sizes=(1,),
                      mode=lax.GatherScatterMode.PROMISE_IN_BOUNDS)


def _agg_body(tables, src_hbm, dst_hbm, vals_hbm, out_hbm,
              src_v, dst_v, vals_v, rows_a, rows_b, zeros_v, accum,
              sem_a, sem_b):
    c = lax.axis_index("c")
    s = lax.axis_index("s")
    wid = c * 16 + s
    row0 = s * 640

    z16 = jnp.zeros((16,), jnp.float32)

    def zfill(i, carry):
        zeros_v[i // 8, pl.ds((i % 8) * 16, 16)] = z16
        return carry

    lax.fori_loop(0, 16 * 8, zfill, 0)

    def one_pass(p, carry):
        def zdma(b, cz):
            pltpu.sync_copy(zeros_v, accum.at[pl.ds(row0 + b * 16, 16)])
            return cz

        lax.fori_loop(0, 40, zdma, 0)
        plsc.subcore_barrier()
        pltpu.sync_copy(src_hbm.at[p, wid], src_v)
        pltpu.sync_copy(dst_hbm.at[p, wid], dst_v)
        pltpu.sync_copy(vals_hbm.at[p, wid], vals_v)

        def process(ch, rows):
            @plsc.parallel_loop(0, _CH // 16)
            def scale(g):
                valv = vals_v[ch, pl.ds(g * 16, 16)]
                for e16 in range(16):
                    vv = _bcast_lane(valv, e16)
                    e = g * 16 + e16
                    for j in range(8):
                        rows[e, pl.ds(j * 16, 16)] = (
                            rows[e, pl.ds(j * 16, 16)] * vv)
            pltpu.sync_copy(rows, accum.at[dst_v.at[ch]], add=True)

        pltpu.async_copy(tables.at[src_v.at[0]], rows_a, sem_a)

        def two_chunks(k, carry2):
            ch0 = 2 * k
            ch1 = ch0 + 1
            pltpu.async_copy(tables.at[src_v.at[ch1]], rows_b, sem_b)
            pltpu.make_async_copy(tables.at[src_v.at[ch0]], rows_a,
                                  sem_a).wait()
            process(ch0, rows_a)

            @pl.when(k < _NCH // 2 - 1)
            def _():
                pltpu.async_copy(tables.at[src_v.at[ch0 + 2]], rows_a, sem_a)

            pltpu.make_async_copy(tables.at[src_v.at[ch1]], rows_b,
                                  sem_b).wait()
            process(ch1, rows_b)
            return carry2

        lax.fori_loop(0, _NCH // 2, two_chunks, 0)
        plsc.subcore_barrier()
        for b in range(5):
            pltpu.sync_copy(
                accum.at[pl.ds(row0 + b * _CH, _CH)],
                out_hbm.at[c, p, pl.ds(row0 + b * _CH, _CH)])
        return carry

    lax.fori_loop(0, _NP, one_pass, 0)


def _dec_body(qcat, vtab, uidx, vidx, cvec, out_hbm,
              uix_v, vix_v, q_a, q_b, v_a, v_b, o_v, c_v,
              sem_qa, sem_qb, sem_va, sem_vb):
    c = lax.axis_index("c")
    s = lax.axis_index("s")
    wid = c * 16 + s
    pltpu.sync_copy(uidx.at[wid], uix_v)
    pltpu.sync_copy(vidx.at[wid], vix_v)
    pltpu.sync_copy(cvec, c_v)
    c0 = c_v[pl.ds(0, 16)]
    c1 = c_v[pl.ds(16, 16)]

    def process(ch, q_v, v_v):
        @plsc.parallel_loop(0, _CH, unroll=2)
        def link(e):
            vs0 = v_v[e, pl.ds(0, 16)]
            acc0 = q_v[e, pl.ds(0, 16)] * vs0
            acc1 = q_v[e, pl.ds(_PF, 16)] * vs0
            for j in range(1, 5):
                vsj = v_v[e, pl.ds(16 * j, 16)]
                acc0 = acc0 + q_v[e, pl.ds(16 * j, 16)] * vsj
                acc1 = acc1 + q_v[e, pl.ds(_PF + 16 * j, 16)] * vsj
            b0 = jnp.sum(acc0)
            b1 = jnp.sum(acc1)
            o_v[e // 8, pl.ds((e % 8) * 16, 16)] = b0 * c0 + b1 * c1

        pltpu.sync_copy(o_v, out_hbm.at[pl.ds(wid * 640 + ch * 16, 16)])

    pltpu.async_copy(qcat.at[uix_v.at[0]], q_a, sem_qa)
    pltpu.async_copy(vtab.at[vix_v.at[0]], v_a, sem_va)

    def two_chunks(k, carry):
        ch0 = 2 * k
        ch1 = ch0 + 1
        pltpu.async_copy(qcat.at[uix_v.at[ch1]], q_b, sem_qb)
        pltpu.async_copy(vtab.at[vix_v.at[ch1]], v_b, sem_vb)
        pltpu.make_async_copy(qcat.at[uix_v.at[ch0]], q_a, sem_qa).wait()
        pltpu.make_async_copy(vtab.at[vix_v.at[ch0]], v_a, sem_va).wait()
        process(ch0, q_a, v_a)

        @pl.when(k < _DNCH // 2 - 1)
        def _():
            pltpu.async_copy(qcat.at[uix_v.at[ch0 + 2]], q_a, sem_qa)
            pltpu.async_copy(vtab.at[vix_v.at[ch0 + 2]], v_a, sem_va)

        pltpu.make_async_copy(qcat.at[uix_v.at[ch1]], q_b, sem_qb).wait()
        pltpu.make_async_copy(vtab.at[vix_v.at[ch1]], v_b, sem_vb).wait()
        process(ch1, q_b, v_b)
        return carry

    lax.fori_loop(0, _DNCH // 2, two_chunks, 0)


@functools.cache
def _build_sc_kernels():
    mesh = plsc.VectorSubcoreMesh(core_axis_name="c", subcore_axis_name="s",
                                  num_cores=2, num_subcores=16)
    params = pltpu.CompilerParams(needs_layout_passes=False)
    agg = pl.kernel(
        _agg_body,
        out_type=jax.ShapeDtypeStruct((2, _NP, _NUP, _PF), jnp.float32),
        mesh=mesh,
        compiler_params=params,
        scratch_types=[
            pltpu.VMEM((_NCH, _CH), jnp.int32),
            pltpu.VMEM((_NCH, _CH), jnp.int32),
            pltpu.VMEM((_NCH, _CH), jnp.float32),
            pltpu.VMEM((_CH, _PF), jnp.float32),
            pltpu.VMEM((_CH, _PF), jnp.float32),
            pltpu.VMEM((16, _PF), jnp.float32),
            pltpu.VMEM_SHARED((_NUP, _PF), jnp.float32),
            pltpu.SemaphoreType.DMA,
            pltpu.SemaphoreType.DMA,
        ],
    )
    dec = pl.kernel(
        _dec_body,
        out_type=jax.ShapeDtypeStruct((_LP // 8, _PF), jnp.float32),
        mesh=mesh,
        compiler_params=params,
        scratch_types=[
            pltpu.VMEM((_DNCH, _CH), jnp.int32),
            pltpu.VMEM((_DNCH, _CH), jnp.int32),
            pltpu.VMEM((_CH, 2 * _PF), jnp.float32),
            pltpu.VMEM((_CH, 2 * _PF), jnp.float32),
            pltpu.VMEM((_CH, _PF), jnp.float32),
            pltpu.VMEM((_CH, _PF), jnp.float32),
            pltpu.VMEM((16, _PF), jnp.float32),
            pltpu.VMEM((32,), jnp.float32),
            pltpu.SemaphoreType.DMA,
            pltpu.SemaphoreType.DMA,
            pltpu.SemaphoreType.DMA,
            pltpu.SemaphoreType.DMA,
        ],
    )
    return agg, dec


def _mm1_body(uv_ref, w_ref, o_ref):
    o_ref[0] = jnp.dot(uv_ref[0], w_ref[0],
                       preferred_element_type=jnp.float32)


_tc_tables = pl.pallas_call(
    _mm1_body,
    grid=(10, 10),
    in_specs=[
        pl.BlockSpec((1, 1000, 128), lambda p, m: (p % 2, m, 0)),
        pl.BlockSpec((1, 128, _PF), lambda p, m: (p, 0, 0)),
    ],
    out_specs=pl.BlockSpec((1, 1000, _PF), lambda p, m: (p, m, 0)),
    out_shape=jax.ShapeDtypeStruct((_NP, _NU, _PF), jnp.float32),
)


def _dense_body(a0_ref, a1_ref, wd_ref, o_ref):
    r = pl.program_id(2)
    z = jnp.maximum(a0_ref[0] + a1_ref[0], 0.0)
    part = jnp.dot(z, wd_ref[0], preferred_element_type=jnp.float32)

    @pl.when(r == 0)
    def _():
        o_ref[0] = part

    @pl.when(r != 0)
    def _():
        o_ref[0] = o_ref[0] + part


_tc_dense = pl.pallas_call(
    _dense_body,
    grid=(2, 10, 5),
    in_specs=[
        pl.BlockSpec((1, 1024, _PF), lambda d, m, r: (2 * r + d, m, 0)),
        pl.BlockSpec((1, 1024, _PF), lambda d, m, r: (2 * r + d, m, 0)),
        pl.BlockSpec((1, _PF, _PF), lambda d, m, r: (r, 0, 0)),
    ],
    out_specs=pl.BlockSpec((1, 1024, _PF), lambda d, m, r: (d, m, 0)),
    out_shape=jax.ShapeDtypeStruct((2, _NUP, _PF), jnp.float32),
)


def _q_body(h_ref, p_ref, o_ref):
    o_ref[...] = jnp.dot(h_ref[...], p_ref[...],
                         preferred_element_type=jnp.float32)


_tc_q = pl.pallas_call(
    _q_body,
    grid=(10,),
    in_specs=[
        pl.BlockSpec((1024, _PF), lambda m: (m, 0)),
        pl.BlockSpec((_PF, 2 * _PF), lambda m: (0, 0)),
    ],
    out_specs=pl.BlockSpec((1024, 2 * _PF), lambda m: (m, 0)),
    out_shape=jax.ShapeDtypeStruct((_NUP, 2 * _PF), jnp.float32),
)


def kernel(u_features, v_features, support_rows, support_cols, support_vals,
           u_indices, v_indices, W_gcn, W_dense, P, coeffs):
    f32 = jnp.float32
    i32 = jnp.int32

    w5 = jnp.pad(W_gcn.reshape(128, 5, 100).transpose(1, 0, 2),
                 ((0, 0), (0, 0), (0, _PF - 100)))
    w_stack = jnp.repeat(w5, 2, axis=0)
    uv = jnp.stack([v_features, u_features])
    wd_stack = jnp.pad(W_dense.reshape(5, 100, 75),
                       ((0, 0), (0, _PF - 100), (0, _PF - 75)))
    p0 = jnp.pad(P[0], ((0, _PF - 75), (0, _PF - 75)))
    p1 = jnp.pad(P[1], ((0, _PF - 75), (0, _PF - 75)))
    pcat = jnp.concatenate([p0, p1], axis=1)
    cvec = jnp.concatenate([
        coeffs[0], jnp.zeros((11,), f32),
        coeffs[1], jnp.zeros((11,), f32)])

    pad_e = (jnp.arange(_EP - 128000, dtype=i32) * 97) % _NU
    zval = jnp.zeros((_EP - 128000,), f32)
    srcs, dsts, valsl = [], [], []
    for r in range(5):
        for d in range(2):
            p = 2 * r + d
            src = support_cols[r] if d == 0 else support_rows[r]
            dst = support_rows[r] if d == 0 else support_cols[r]
            srcs.append(jnp.concatenate([src, pad_e]) + p * _NU)
            dsts.append(jnp.concatenate([dst, pad_e]))
            valsl.append(jnp.concatenate([support_vals[r], zval]))
    src_hbm = jnp.stack(srcs).reshape(_NP, 32, _NCH, _CH)
    dst_hbm = jnp.stack(dsts).reshape(_NP, 32, _NCH, _CH)
    vals_hbm = jnp.stack(valsl).reshape(_NP, 32, _NCH, _CH)

    pad_l = (jnp.arange(_LP - 160000, dtype=i32) * 131) % _NU
    uix = jnp.concatenate([u_indices, pad_l]).reshape(32, _DNCH, _CH)
    vix = jnp.concatenate([v_indices, pad_l]).reshape(32, _DNCH, _CH)

    sc_aggregate, sc_decoder = _build_sc_kernels()
    tables = _tc_tables(uv, w_stack).reshape(_NP * _NU, _PF)
    aggs = sc_aggregate(tables, src_hbm, dst_hbm, vals_hbm)
    h = _tc_dense(aggs[0], aggs[1], wd_stack)
    qcat = _tc_q(h[0], pcat)
    out = sc_decoder(qcat, h[1], uix, vix, cvec)
    return out.reshape(_LP, 16)[:160000, :5]
